# baseline (speedup 1.0000x reference)
import functools

import jax
import jax.numpy as jnp
from jax import lax
from jax.experimental import pallas as pl
from jax.experimental.pallas import tpu as pltpu
from jax.experimental.pallas import tpu_sc as plsc

NC = 2
NS = 16
NW = NC * NS
CH = 128
LANES = 16


def _make_deg(n_pad, nblk):
    rows_pt = n_pad // NS
    mesh = plsc.VectorSubcoreMesh(core_axis_name="c", subcore_axis_name="s")

    @functools.partial(
        pl.kernel, mesh=mesh,
        compiler_params=pltpu.CompilerParams(use_tc_tiling_on_sc=False),
        out_type=jax.ShapeDtypeStruct((NC * n_pad,), jnp.float32),
        scratch_types=[
            pltpu.VMEM((nblk, CH), jnp.int32),
            pltpu.VMEM((CH,), jnp.float32),
            pltpu.VMEM((rows_pt,), jnp.float32),
            pltpu.MemorySpace.VMEM_SHARED((n_pad,), jnp.float32),
            pltpu.SemaphoreType.DMA,
        ],
    )
    def deg(dst_hbm, out_hbm, dstb, ones, zbuf, acc, sem):
        cid = lax.axis_index("c")
        sid = lax.axis_index("s")
        wid = sid * NC + cid
        r0 = sid * rows_pt
        zv = jnp.zeros((LANES,), jnp.float32)
        ov = jnp.ones((LANES,), jnp.float32)
        for i in range(CH // LANES):
            ones[pl.ds(i * LANES, LANES)] = ov

        def zbody(i, c):
            zbuf[pl.ds(i * LANES, LANES)] = zv
            return c

        lax.fori_loop(0, rows_pt // LANES, zbody, 0)
        pltpu.sync_copy(zbuf, acc.at[pl.ds(r0, rows_pt)])
        plsc.subcore_barrier()
        pltpu.sync_copy(dst_hbm.at[wid], dstb)

        def body(j, c):
            pltpu.async_copy(ones, acc.at[dstb.at[j]], sem, add=True)
            return c

        lax.fori_loop(0, nblk, body, 0)

        def dbody(j, c):
            pltpu.make_async_copy(ones, acc.at[dstb.at[j]], sem).wait()
            return c

        lax.fori_loop(0, nblk, dbody, 0)
        plsc.subcore_barrier()
        pltpu.sync_copy(acc.at[pl.ds(r0, rows_pt)],
                        out_hbm.at[pl.ds(cid * n_pad + r0, rows_pt)])

    return deg


def _make_agg(n_pad, hid, nblk):
    rows_pt = n_pad // NS
    mesh = plsc.VectorSubcoreMesh(core_axis_name="c", subcore_axis_name="s")

    @functools.partial(
        pl.kernel, mesh=mesh,
        compiler_params=pltpu.CompilerParams(use_tc_tiling_on_sc=False),
        out_type=jax.ShapeDtypeStruct((NC, n_pad, hid), jnp.float32),
        scratch_types=[
            pltpu.VMEM((nblk, CH), jnp.int32),
            pltpu.VMEM((nblk, CH), jnp.int32),
            pltpu.VMEM((CH, hid), jnp.float32),
            pltpu.VMEM((CH, hid), jnp.float32),
            pltpu.VMEM((CH, hid), jnp.float32),
            pltpu.VMEM((CH, hid), jnp.float32),
            pltpu.VMEM((rows_pt, hid), jnp.float32),
            pltpu.MemorySpace.VMEM_SHARED((n_pad, hid), jnp.float32),
            pltpu.SemaphoreType.DMA,
            pltpu.SemaphoreType.DMA,
        ],
    )
    def agg(src_hbm, dst_hbm, hp_hbm, out_hbm, srcb, dstb, msg0, msg1, msg2,
            msg3, zbuf, acc, sem_g, sem_s):
        cid = lax.axis_index("c")
        sid = lax.axis_index("s")
        wid = sid * NC + cid
        r0 = sid * rows_pt
        zv = jnp.zeros((LANES,), jnp.float32)

        def zbody(i, c):
            zbuf[i, :] = zv
            return c

        lax.fori_loop(0, rows_pt, zbody, 0)
        pltpu.sync_copy(zbuf, acc.at[pl.ds(r0, rows_pt)])
        plsc.subcore_barrier()
        pltpu.sync_copy(src_hbm.at[wid], srcb)
        pltpu.sync_copy(dst_hbm.at[wid], dstb)

        msg = (msg0, msg1, msg2, msg3)
        ring = len(msg)

        def chunk(j, b):
            @pl.when(j >= ring)
            def _():
                pltpu.make_async_copy(msg[b], acc.at[dstb.at[j - ring]],
                                      sem_s).wait()

            pltpu.async_copy(hp_hbm.at[srcb.at[j]], msg[b], sem_g).wait()
            pltpu.async_copy(msg[b], acc.at[dstb.at[j]], sem_s, add=True)

        def body(g, c):
            for b in range(ring):
                chunk(g * ring + b, b)
            return c

        lax.fori_loop(0, nblk // ring, body, 0)
        for j in range((nblk // ring) * ring, nblk):
            chunk(j, j % ring)
        for j in range(max(0, nblk - ring), nblk):
            pltpu.make_async_copy(msg[j % ring], acc.at[dstb.at[j]],
                                  sem_s).wait()
        plsc.subcore_barrier()
        pltpu.sync_copy(acc.at[pl.ds(r0, rows_pt)],
                        out_hbm.at[cid, pl.ds(r0, rows_pt)])

    return agg


_BLK = 2000


def _tc_mm(x, w):
    n, f_in = x.shape
    hid = w.shape[1]

    def body(x_ref, w_ref, o_ref):
        o_ref[...] = jnp.dot(x_ref[...], w_ref[...],
                             preferred_element_type=jnp.float32)

    return pl.pallas_call(
        body,
        grid=(n // _BLK,),
        in_specs=[
            pl.BlockSpec((_BLK, f_in), lambda i: (i, 0)),
            pl.BlockSpec((f_in, hid), lambda i: (0, 0)),
        ],
        out_specs=pl.BlockSpec((_BLK, hid), lambda i: (i, 0)),
        out_shape=jax.ShapeDtypeStruct((n, hid), jnp.float32),
    )(x, w)


def _tc_scale(d0, d1, xw1):
    n, hid = xw1.shape

    def body(d0_ref, d1_ref, xw_ref, dinv_ref, hp_ref):
        dinv = lax.rsqrt(d0_ref[...] + d1_ref[...] + 1.0)
        dinv_ref[...] = dinv
        hp_ref[...] = xw_ref[...] * dinv

    return pl.pallas_call(
        body,
        grid=(n // _BLK,),
        in_specs=[
            pl.BlockSpec((_BLK, 1), lambda i: (i, 0)),
            pl.BlockSpec((_BLK, 1), lambda i: (i, 0)),
            pl.BlockSpec((_BLK, hid), lambda i: (i, 0)),
        ],
        out_specs=(pl.BlockSpec((_BLK, 1), lambda i: (i, 0)),
                   pl.BlockSpec((_BLK, hid), lambda i: (i, 0))),
        out_shape=(jax.ShapeDtypeStruct((n, 1), jnp.float32),
                   jax.ShapeDtypeStruct((n, hid), jnp.float32)),
    )(d0, d1, xw1)


def _tc_layer1(p, hp1, dinv, b1):
    n, hid = hp1.shape

    def body(p_ref, hp_ref, dinv_ref, b_ref, h_ref, hh_ref):
        agg = p_ref[0] + p_ref[1] + hp_ref[...]
        h = jax.nn.relu(agg * dinv_ref[...] + b_ref[...])
        h_ref[...] = h
        hh_ref[...] = h * dinv_ref[...]

    return pl.pallas_call(
        body,
        grid=(n // _BLK,),
        in_specs=[
            pl.BlockSpec((2, _BLK, hid), lambda i: (0, i, 0)),
            pl.BlockSpec((_BLK, hid), lambda i: (i, 0)),
            pl.BlockSpec((_BLK, 1), lambda i: (i, 0)),
            pl.BlockSpec((1, hid), lambda i: (0, 0)),
        ],
        out_specs=(pl.BlockSpec((_BLK, hid), lambda i: (i, 0)),
                   pl.BlockSpec((_BLK, hid), lambda i: (i, 0))),
        out_shape=(jax.ShapeDtypeStruct((n, hid), jnp.float32),
                   jax.ShapeDtypeStruct((n, hid), jnp.float32)),
    )(p, hp1, dinv, b1)


def _tc_layer2(q, hh, dinv, w2, b2):
    n, hid = hh.shape
    ncls = w2.shape[1]

    def body(q_ref, hh_ref, dinv_ref, w_ref, b_ref, o_ref):
        agg = q_ref[0] + q_ref[1] + hh_ref[...]
        o_ref[...] = (jnp.dot(agg, w_ref[...],
                              preferred_element_type=jnp.float32)
                      * dinv_ref[...] + b_ref[...])

    return pl.pallas_call(
        body,
        grid=(n // _BLK,),
        in_specs=[
            pl.BlockSpec((2, _BLK, hid), lambda i: (0, i, 0)),
            pl.BlockSpec((_BLK, hid), lambda i: (i, 0)),
            pl.BlockSpec((_BLK, 1), lambda i: (i, 0)),
            pl.BlockSpec((hid, ncls), lambda i: (0, 0)),
            pl.BlockSpec((1, ncls), lambda i: (0, 0)),
        ],
        out_specs=pl.BlockSpec((_BLK, ncls), lambda i: (i, 0)),
        out_shape=jax.ShapeDtypeStruct((n, ncls), jnp.float32),
    )(q, hh, dinv, w2, b2)


def kernel(x, edge_index, W1, b1, W2, b2):
    n, f_in = x.shape
    hid = W1.shape[1]
    ncls = W2.shape[1]
    e = edge_index.shape[1]
    assert hid == LANES

    rows_pt = -(-n // NS)
    rows_pt = -(-rows_pt // 128) * 128
    n_pad = rows_pt * NS

    e_blk = NW * CH
    nblk = -(-e // e_blk)
    e_pad = nblk * e_blk

    src = edge_index[0]
    dst = edge_index[1]
    if e_pad != e:
        npad_fill = jnp.arange(e_pad - e, dtype=jnp.int32)
        src = jnp.concatenate([src, npad_fill % n])
        dst = jnp.concatenate([dst, n + npad_fill % (n_pad - n)])
    src_r = src.reshape(NW, nblk, CH)
    dst_r = dst.reshape(NW, nblk, CH)

    degp = _make_deg(n_pad, nblk)(dst_r).reshape(NC, n_pad)
    xw1 = _tc_mm(x, W1)
    d0 = degp[0][:n, None]
    d1 = degp[1][:n, None]
    dinv, hp1 = _tc_scale(d0, d1, xw1)

    agg = _make_agg(n_pad, hid, nblk)
    p = agg(src_r, dst_r, hp1)
    h, hh = _tc_layer1(p, hp1, dinv, b1.reshape(1, hid))
    q = agg(src_r, dst_r, hh)
    out = _tc_layer2(q, hh, dinv, W2, b2.reshape(1, ncls))

    return (out, h)


# speedup vs baseline: 1.0174x; 1.0174x over previous
import functools

import jax
import jax.numpy as jnp
from jax import lax
from jax.experimental import pallas as pl
from jax.experimental.pallas import tpu as pltpu
from jax.experimental.pallas import tpu_sc as plsc

NC = 2
NS = 16
NW = NC * NS
CH = 128
LANES = 16


def _make_deg(n_pad, nblk):
    rows_pt = n_pad // NS
    mesh = plsc.VectorSubcoreMesh(core_axis_name="c", subcore_axis_name="s")

    @functools.partial(
        pl.kernel, mesh=mesh,
        compiler_params=pltpu.CompilerParams(use_tc_tiling_on_sc=False),
        out_type=jax.ShapeDtypeStruct((NC * n_pad,), jnp.float32),
        scratch_types=[
            pltpu.VMEM((nblk, CH), jnp.int32),
            pltpu.VMEM((CH,), jnp.float32),
            pltpu.VMEM((rows_pt,), jnp.float32),
            pltpu.MemorySpace.VMEM_SHARED((n_pad,), jnp.float32),
            pltpu.SemaphoreType.DMA,
        ],
    )
    def deg(dst_hbm, out_hbm, dstb, ones, zbuf, acc, sem):
        cid = lax.axis_index("c")
        sid = lax.axis_index("s")
        wid = sid * NC + cid
        r0 = sid * rows_pt
        idx_cp = pltpu.async_copy(dst_hbm.at[wid], dstb, sem)
        zv = jnp.zeros((LANES,), jnp.float32)
        ov = jnp.ones((LANES,), jnp.float32)
        for i in range(CH // LANES):
            ones[pl.ds(i * LANES, LANES)] = ov

        def zbody(i, c):
            zbuf[pl.ds(i * LANES, LANES)] = zv
            return c

        lax.fori_loop(0, rows_pt // LANES, zbody, 0)
        pltpu.sync_copy(zbuf, acc.at[pl.ds(r0, rows_pt)])
        plsc.subcore_barrier()
        idx_cp.wait()

        def body(j, c):
            pltpu.async_copy(ones, acc.at[dstb.at[j]], sem, add=True)
            return c

        lax.fori_loop(0, nblk, body, 0)

        def dbody(j, c):
            pltpu.make_async_copy(ones, acc.at[dstb.at[j]], sem).wait()
            return c

        lax.fori_loop(0, nblk, dbody, 0)
        plsc.subcore_barrier()
        pltpu.sync_copy(acc.at[pl.ds(r0, rows_pt)],
                        out_hbm.at[pl.ds(cid * n_pad + r0, rows_pt)])

    return deg


def _make_agg(n_pad, hid, nblk):
    rows_pt = n_pad // NS
    mesh = plsc.VectorSubcoreMesh(core_axis_name="c", subcore_axis_name="s")

    @functools.partial(
        pl.kernel, mesh=mesh,
        compiler_params=pltpu.CompilerParams(use_tc_tiling_on_sc=False),
        out_type=jax.ShapeDtypeStruct((NC, n_pad, hid), jnp.float32),
        scratch_types=[
            pltpu.VMEM((nblk, CH), jnp.int32),
            pltpu.VMEM((nblk, CH), jnp.int32),
            pltpu.VMEM((CH, hid), jnp.float32),
            pltpu.VMEM((CH, hid), jnp.float32),
            pltpu.VMEM((CH, hid), jnp.float32),
            pltpu.VMEM((CH, hid), jnp.float32),
            pltpu.VMEM((rows_pt, hid), jnp.float32),
            pltpu.MemorySpace.VMEM_SHARED((n_pad, hid), jnp.float32),
            pltpu.SemaphoreType.DMA,
            pltpu.SemaphoreType.DMA,
        ],
    )
    def agg(src_hbm, dst_hbm, hp_hbm, out_hbm, srcb, dstb, msg0, msg1, msg2,
            msg3, zbuf, acc, sem_g, sem_s):
        cid = lax.axis_index("c")
        sid = lax.axis_index("s")
        wid = sid * NC + cid
        r0 = sid * rows_pt
        src_cp = pltpu.async_copy(src_hbm.at[wid], srcb, sem_g)
        dst_cp = pltpu.async_copy(dst_hbm.at[wid], dstb, sem_g)
        zv = jnp.zeros((LANES,), jnp.float32)

        def zbody(i, c):
            zbuf[i, :] = zv
            return c

        lax.fori_loop(0, rows_pt, zbody, 0)
        pltpu.sync_copy(zbuf, acc.at[pl.ds(r0, rows_pt)])
        plsc.subcore_barrier()
        src_cp.wait()
        dst_cp.wait()

        msg = (msg0, msg1, msg2, msg3)
        ring = len(msg)

        def chunk(j, b):
            @pl.when(j >= ring)
            def _():
                pltpu.make_async_copy(msg[b], acc.at[dstb.at[j - ring]],
                                      sem_s).wait()

            pltpu.async_copy(hp_hbm.at[srcb.at[j]], msg[b], sem_g).wait()
            pltpu.async_copy(msg[b], acc.at[dstb.at[j]], sem_s, add=True)

        def body(g, c):
            for b in range(ring):
                chunk(g * ring + b, b)
            return c

        lax.fori_loop(0, nblk // ring, body, 0)
        for j in range((nblk // ring) * ring, nblk):
            chunk(j, j % ring)
        for j in range(max(0, nblk - ring), nblk):
            pltpu.make_async_copy(msg[j % ring], acc.at[dstb.at[j]],
                                  sem_s).wait()
        plsc.subcore_barrier()
        pltpu.sync_copy(acc.at[pl.ds(r0, rows_pt)],
                        out_hbm.at[cid, pl.ds(r0, rows_pt)])

    return agg


_BLK = 2000


def _tc_mm(x, w):
    n, f_in = x.shape
    hid = w.shape[1]

    def body(x_ref, w_ref, o_ref):
        o_ref[...] = jnp.dot(x_ref[...], w_ref[...],
                             preferred_element_type=jnp.float32)

    return pl.pallas_call(
        body,
        grid=(n // _BLK,),
        in_specs=[
            pl.BlockSpec((_BLK, f_in), lambda i: (i, 0)),
            pl.BlockSpec((f_in, hid), lambda i: (0, 0)),
        ],
        out_specs=pl.BlockSpec((_BLK, hid), lambda i: (i, 0)),
        out_shape=jax.ShapeDtypeStruct((n, hid), jnp.float32),
    )(x, w)


def _tc_scale(d0, d1, xw1):
    n, hid = xw1.shape

    def body(d0_ref, d1_ref, xw_ref, dinv_ref, hp_ref):
        dinv = lax.rsqrt(d0_ref[...] + d1_ref[...] + 1.0)
        dinv_ref[...] = dinv
        hp_ref[...] = xw_ref[...] * dinv

    return pl.pallas_call(
        body,
        grid=(n // _BLK,),
        in_specs=[
            pl.BlockSpec((_BLK, 1), lambda i: (i, 0)),
            pl.BlockSpec((_BLK, 1), lambda i: (i, 0)),
            pl.BlockSpec((_BLK, hid), lambda i: (i, 0)),
        ],
        out_specs=(pl.BlockSpec((_BLK, 1), lambda i: (i, 0)),
                   pl.BlockSpec((_BLK, hid), lambda i: (i, 0))),
        out_shape=(jax.ShapeDtypeStruct((n, 1), jnp.float32),
                   jax.ShapeDtypeStruct((n, hid), jnp.float32)),
    )(d0, d1, xw1)


def _tc_layer1(p, hp1, dinv, b1):
    n, hid = hp1.shape

    def body(p_ref, hp_ref, dinv_ref, b_ref, h_ref, hh_ref):
        agg = p_ref[0] + p_ref[1] + hp_ref[...]
        h = jax.nn.relu(agg * dinv_ref[...] + b_ref[...])
        h_ref[...] = h
        hh_ref[...] = h * dinv_ref[...]

    return pl.pallas_call(
        body,
        grid=(n // _BLK,),
        in_specs=[
            pl.BlockSpec((2, _BLK, hid), lambda i: (0, i, 0)),
            pl.BlockSpec((_BLK, hid), lambda i: (i, 0)),
            pl.BlockSpec((_BLK, 1), lambda i: (i, 0)),
            pl.BlockSpec((1, hid), lambda i: (0, 0)),
        ],
        out_specs=(pl.BlockSpec((_BLK, hid), lambda i: (i, 0)),
                   pl.BlockSpec((_BLK, hid), lambda i: (i, 0))),
        out_shape=(jax.ShapeDtypeStruct((n, hid), jnp.float32),
                   jax.ShapeDtypeStruct((n, hid), jnp.float32)),
    )(p, hp1, dinv, b1)


def _tc_layer2(q, hh, dinv, w2, b2):
    n, hid = hh.shape
    ncls = w2.shape[1]

    def body(q_ref, hh_ref, dinv_ref, w_ref, b_ref, o_ref):
        agg = q_ref[0] + q_ref[1] + hh_ref[...]
        o_ref[...] = (jnp.dot(agg, w_ref[...],
                              preferred_element_type=jnp.float32)
                      * dinv_ref[...] + b_ref[...])

    return pl.pallas_call(
        body,
        grid=(n // _BLK,),
        in_specs=[
            pl.BlockSpec((2, _BLK, hid), lambda i: (0, i, 0)),
            pl.BlockSpec((_BLK, hid), lambda i: (i, 0)),
            pl.BlockSpec((_BLK, 1), lambda i: (i, 0)),
            pl.BlockSpec((hid, ncls), lambda i: (0, 0)),
            pl.BlockSpec((1, ncls), lambda i: (0, 0)),
        ],
        out_specs=pl.BlockSpec((_BLK, ncls), lambda i: (i, 0)),
        out_shape=jax.ShapeDtypeStruct((n, ncls), jnp.float32),
    )(q, hh, dinv, w2, b2)


def kernel(x, edge_index, W1, b1, W2, b2):
    n, f_in = x.shape
    hid = W1.shape[1]
    ncls = W2.shape[1]
    e = edge_index.shape[1]
    assert hid == LANES

    rows_pt = -(-n // NS)
    rows_pt = -(-rows_pt // 128) * 128
    n_pad = rows_pt * NS

    e_blk = NW * CH
    nblk = -(-e // e_blk)
    e_pad = nblk * e_blk

    src = edge_index[0]
    dst = edge_index[1]
    if e_pad != e:
        npad_fill = jnp.arange(e_pad - e, dtype=jnp.int32)
        src = jnp.concatenate([src, npad_fill % n])
        dst = jnp.concatenate([dst, n + npad_fill % (n_pad - n)])
    src_r = src.reshape(NW, nblk, CH)
    dst_r = dst.reshape(NW, nblk, CH)

    degp = _make_deg(n_pad, nblk)(dst_r).reshape(NC, n_pad)
    xw1 = _tc_mm(x, W1)
    d0 = degp[0][:n, None]
    d1 = degp[1][:n, None]
    dinv, hp1 = _tc_scale(d0, d1, xw1)

    agg = _make_agg(n_pad, hid, nblk)
    p = agg(src_r, dst_r, hp1)
    h, hh = _tc_layer1(p, hp1, dinv, b1.reshape(1, hid))
    q = agg(src_r, dst_r, hh)
    out = _tc_layer2(q, hh, dinv, W2, b2.reshape(1, ncls))

    return (out, h)
